# baseline (speedup 1.0000x reference)
import functools

import jax
import jax.numpy as jnp
import numpy as np
from jax import lax
from jax.experimental import pallas as pl
from jax.experimental.pallas import tpu as pltpu
from jax.experimental.pallas import tpu_sc as plsc

_D = 1024
_B = 16
_L = 512
_R = 256
_NCHUNK = 16
_NBUF = 3


def _pos_encoding(d_model, seq_len=1000):
    position = np.arange(seq_len, dtype=np.float32).reshape(-1, 1)
    i = np.arange(d_model) // 2
    exp_term = 2.0 * i / float(d_model)
    div_term = np.power(10000.0, exp_term).reshape(1, -1).astype(np.float32)
    pe = position / div_term
    pe[:, 0::2] = np.sin(pe[:, 0::2])
    pe[:, 1::2] = np.cos(pe[:, 1::2])
    return pe


_POS = jnp.asarray(_pos_encoding(_D)[: _L + 1], dtype=jnp.float32)


def _lane_bcast(v, r):
    idx = jnp.full((16,), r, dtype=jnp.int32)
    dnums = lax.GatherDimensionNumbers(
        offset_dims=(), collapsed_slice_dims=(0,), start_index_map=(0,)
    )
    return lax.gather(
        v, idx[:, None], dnums, (1,),
        mode=lax.GatherScatterMode.PROMISE_IN_BOUNDS,
    )


def _sc_body(data_h, mask_h, rv_h, pm_h, pos_h, out_h,
             rv_v, pm_v, mask_v,
             rows0_v, rows1_v, rows2_v, pos0_v, pos1_v, pos2_v,
             semd0, semd1, semd2, semp0, semp1, semp2,
             semw0, semw1, semw2):
    cid = lax.axis_index("c")
    sid = lax.axis_index("s")
    w = sid * 2 + cid
    b = lax.shift_right_logical(w, 1)
    half = lax.bitwise_and(w, 1)
    t0w = half * (_L // 2)

    data_b = data_h.at[b]
    out_b = out_h.at[b]

    pltpu.sync_copy(rv_h.at[b], rv_v)
    pltpu.sync_copy(pm_h.at[b], pm_v)
    pltpu.sync_copy(mask_h, mask_v)

    iota = lax.iota(jnp.int32, 16)

    rows = (rows0_v, rows1_v, rows2_v)
    poss = (pos0_v, pos1_v, pos2_v)
    semd = (semd0, semd1, semd2)
    semp = (semp0, semp1, semp2)
    semw = (semw0, semw1, semw2)

    def indices_for(t):
        rv = plsc.load_gather(rv_v, [jnp.maximum(t - 1, 0)])
        j = jnp.where(t == 0, 0, rv + 1)
        pmv = plsc.load_gather(pm_v, [jnp.clip(j - 1, 0, _R - 1)])
        valid = (j == 0) | ((j <= _R) & (pmv == 1))
        src = jnp.where(valid, j, 0)
        return src, valid

    def issue_gathers(t, bi, tbase=None):
        src, valid = indices_for(t)
        cpd = pltpu.async_copy(data_b.at[src], rows[bi], semd[bi])
        if tbase is None:
            cpp = pltpu.async_copy(pos_h.at[t], poss[bi], semp[bi])
        else:
            cpp = pltpu.async_copy(pos_h.at[pl.ds(tbase, 16)], poss[bi], semp[bi])
        return valid, cpd, cpp

    def compute(valid, bi):
        rv_, pv_ = rows[bi], poss[bi]
        vf = jnp.where(valid, 1, 0)
        vbc = [_lane_bcast(vf, r) != 0 for r in range(16)]

        def slice_body(s, _):
            off = s * 16
            m = mask_v[pl.ds(off, 16)]
            for r in range(16):
                g = rv_[r, pl.ds(off, 16)]
                p = pv_[r, pl.ds(off, 16)]
                rv_[r, pl.ds(off, 16)] = jnp.where(vbc[r], g, m) + p
            return 0

        lax.fori_loop(0, _D // 16, slice_body, 0)

    pend = [None] * _NCHUNK
    wr = [None] * _NCHUNK
    pend[0] = issue_gathers(t0w + iota, 0, t0w)
    for k in range(_NCHUNK):
        cur = k % _NBUF
        if k + 1 < _NCHUNK:
            if k >= 2:
                wr[k - 2].wait()
            pend[k + 1] = issue_gathers(
                t0w + 16 * (k + 1) + iota, (k + 1) % _NBUF, t0w + 16 * (k + 1)
            )
        valid, cpd, cpp = pend[k]
        cpd.wait()
        cpp.wait()
        compute(valid, cur)
        wr[k] = pltpu.async_copy(
            rows[cur], out_b.at[pl.ds(t0w + 16 * k, 16)], semw[cur]
        )

    wr[_NCHUNK - 3].wait()
    wr[_NCHUNK - 2].wait()
    wr[_NCHUNK - 1].wait()

    @pl.when(half == 1)
    def _():
        t = jnp.full((16,), _L, jnp.int32)
        valid, cpd, cpp = issue_gathers(t, 0)
        cpd.wait()
        cpp.wait()
        compute(valid, 0)
        pltpu.sync_copy(rows0_v.at[pl.ds(0, 1)], out_b.at[pl.ds(_L, 1)])


@functools.partial(jax.jit, static_argnames=())
def _run(data, mask_token, revert_idx, padding_mask, pos):
    mesh = plsc.VectorSubcoreMesh(core_axis_name="c", subcore_axis_name="s")
    return pl.kernel(
        _sc_body,
        out_type=jax.ShapeDtypeStruct((_B, _L + 1, _D), jnp.float32),
        mesh=mesh,
        compiler_params=pltpu.CompilerParams(needs_layout_passes=False),
        scratch_types=[
            pltpu.VMEM((_L,), jnp.int32),
            pltpu.VMEM((_R,), jnp.int32),
            pltpu.VMEM((_D,), jnp.float32),
            pltpu.VMEM((16, _D), jnp.float32),
            pltpu.VMEM((16, _D), jnp.float32),
            pltpu.VMEM((16, _D), jnp.float32),
            pltpu.VMEM((16, _D), jnp.float32),
            pltpu.VMEM((16, _D), jnp.float32),
            pltpu.VMEM((16, _D), jnp.float32),
            pltpu.SemaphoreType.DMA,
            pltpu.SemaphoreType.DMA,
            pltpu.SemaphoreType.DMA,
            pltpu.SemaphoreType.DMA,
            pltpu.SemaphoreType.DMA,
            pltpu.SemaphoreType.DMA,
            pltpu.SemaphoreType.DMA,
            pltpu.SemaphoreType.DMA,
            pltpu.SemaphoreType.DMA,
        ],
    )(data, mask_token, revert_idx, padding_mask, pos)


def kernel(data, mask_token, revert_idx, device, padding_mask):
    del device
    return _run(data, mask_token, revert_idx, padding_mask, _POS)


# speedup vs baseline: 1.0402x; 1.0402x over previous
import functools

import jax
import jax.numpy as jnp
import numpy as np
from jax import lax
from jax.experimental import pallas as pl
from jax.experimental.pallas import tpu as pltpu
from jax.experimental.pallas import tpu_sc as plsc

_D = 1024
_B = 16
_L = 512
_R = 256
_NCHUNK = 16
_NBUF = 3


def _pos_encoding(d_model, seq_len=1000):
    position = np.arange(seq_len, dtype=np.float32).reshape(-1, 1)
    i = np.arange(d_model) // 2
    exp_term = 2.0 * i / float(d_model)
    div_term = np.power(10000.0, exp_term).reshape(1, -1).astype(np.float32)
    pe = position / div_term
    pe[:, 0::2] = np.sin(pe[:, 0::2])
    pe[:, 1::2] = np.cos(pe[:, 1::2])
    return pe


_POS = jnp.asarray(_pos_encoding(_D)[: _L + 1], dtype=jnp.float32)


def _lane_bcast(v, r):
    idx = jnp.full((16,), r, dtype=jnp.int32)
    dnums = lax.GatherDimensionNumbers(
        offset_dims=(), collapsed_slice_dims=(0,), start_index_map=(0,)
    )
    return lax.gather(
        v, idx[:, None], dnums, (1,),
        mode=lax.GatherScatterMode.PROMISE_IN_BOUNDS,
    )


def _sc_body(data_h, mask_h, rv_h, pm_h, pos_h, out_h,
             rv_v, pm_v, mask_v, pos_sh,
             rows0_v, rows1_v, rows2_v, pos0_v, pos1_v, pos2_v,
             semd0, semd1, semd2, semp0, semp1, semp2,
             semw0, semw1, semw2):
    cid = lax.axis_index("c")
    sid = lax.axis_index("s")
    b = sid
    t0w = cid * (_L // 2)

    data_b = data_h.at[b]
    out_b = out_h.at[b]

    pltpu.sync_copy(rv_h.at[b], rv_v)
    pltpu.sync_copy(pm_h.at[b], pm_v)
    pltpu.sync_copy(mask_h, mask_v)

    pltpu.sync_copy(
        pos_h.at[pl.ds(t0w + sid * 16, 16)], pos_sh.at[pl.ds(sid * 16, 16)]
    )
    plsc.subcore_barrier()

    iota = lax.iota(jnp.int32, 16)

    rows = (rows0_v, rows1_v, rows2_v)
    poss = (pos0_v, pos1_v, pos2_v)
    semd = (semd0, semd1, semd2)
    semp = (semp0, semp1, semp2)
    semw = (semw0, semw1, semw2)

    def indices_for(t):
        rv = plsc.load_gather(rv_v, [jnp.maximum(t - 1, 0)])
        j = jnp.where(t == 0, 0, rv + 1)
        pmv = plsc.load_gather(pm_v, [jnp.clip(j - 1, 0, _R - 1)])
        valid = (j == 0) | ((j <= _R) & (pmv == 1))
        src = jnp.where(valid, j, 0)
        return src, valid

    def issue_gathers(k, bi):
        src, valid = indices_for(t0w + 16 * k + iota)
        cpd = pltpu.async_copy(data_b.at[src], rows[bi], semd[bi])
        cpp = pltpu.async_copy(pos_sh.at[pl.ds(16 * k, 16)], poss[bi], semp[bi])
        return valid, cpd, cpp

    def compute(valid, bi):
        rv_, pv_ = rows[bi], poss[bi]
        vf = jnp.where(valid, 1, 0)
        vbc = [_lane_bcast(vf, r) != 0 for r in range(16)]

        def slice_body(s, _):
            off = s * 16
            m = mask_v[pl.ds(off, 16)]
            for r in range(16):
                g = rv_[r, pl.ds(off, 16)]
                p = pv_[r, pl.ds(off, 16)]
                rv_[r, pl.ds(off, 16)] = jnp.where(vbc[r], g, m) + p
            return 0

        lax.fori_loop(0, _D // 16, slice_body, 0)

    pend = [None] * _NCHUNK
    wr = [None] * _NCHUNK
    pend[0] = issue_gathers(0, 0)
    for k in range(_NCHUNK):
        cur = k % _NBUF
        if k + 1 < _NCHUNK:
            if k >= 2:
                wr[k - 2].wait()
            pend[k + 1] = issue_gathers(k + 1, (k + 1) % _NBUF)
        valid, cpd, cpp = pend[k]
        cpd.wait()
        cpp.wait()
        compute(valid, cur)
        wr[k] = pltpu.async_copy(
            rows[cur], out_b.at[pl.ds(t0w + 16 * k, 16)], semw[cur]
        )

    wr[_NCHUNK - 3].wait()
    wr[_NCHUNK - 2].wait()
    wr[_NCHUNK - 1].wait()

    @pl.when(cid == 1)
    def _():
        t = jnp.full((16,), _L, jnp.int32)
        src, valid = indices_for(t)
        cpd = pltpu.async_copy(data_b.at[src], rows0_v, semd0)
        cpp = pltpu.async_copy(pos_h.at[t], pos0_v, semp0)
        cpd.wait()
        cpp.wait()
        compute(valid, 0)
        pltpu.sync_copy(rows0_v.at[pl.ds(0, 1)], out_b.at[pl.ds(_L, 1)])


@functools.partial(jax.jit, static_argnames=())
def _run(data, mask_token, revert_idx, padding_mask, pos):
    mesh = plsc.VectorSubcoreMesh(core_axis_name="c", subcore_axis_name="s")
    return pl.kernel(
        _sc_body,
        out_type=jax.ShapeDtypeStruct((_B, _L + 1, _D), jnp.float32),
        mesh=mesh,
        compiler_params=pltpu.CompilerParams(needs_layout_passes=False),
        scratch_types=[
            pltpu.VMEM((_L,), jnp.int32),
            pltpu.VMEM((_R,), jnp.int32),
            pltpu.VMEM((_D,), jnp.float32),
            pltpu.VMEM_SHARED((_L // 2, _D), jnp.float32),
            pltpu.VMEM((16, _D), jnp.float32),
            pltpu.VMEM((16, _D), jnp.float32),
            pltpu.VMEM((16, _D), jnp.float32),
            pltpu.VMEM((16, _D), jnp.float32),
            pltpu.VMEM((16, _D), jnp.float32),
            pltpu.VMEM((16, _D), jnp.float32),
            pltpu.SemaphoreType.DMA,
            pltpu.SemaphoreType.DMA,
            pltpu.SemaphoreType.DMA,
            pltpu.SemaphoreType.DMA,
            pltpu.SemaphoreType.DMA,
            pltpu.SemaphoreType.DMA,
            pltpu.SemaphoreType.DMA,
            pltpu.SemaphoreType.DMA,
            pltpu.SemaphoreType.DMA,
        ],
    )(data, mask_token, revert_idx, padding_mask, pos)


def kernel(data, mask_token, revert_idx, device, padding_mask):
    del device
    return _run(data, mask_token, revert_idx, padding_mask, _POS)


# speedup vs baseline: 1.0457x; 1.0054x over previous
import functools

import jax
import jax.numpy as jnp
import numpy as np
from jax import lax
from jax.experimental import pallas as pl
from jax.experimental.pallas import tpu as pltpu
from jax.experimental.pallas import tpu_sc as plsc

_D = 1024
_B = 16
_L = 512
_R = 256
_NCHUNK = 16
_NBUF = 3


def _pos_encoding(d_model, seq_len=1000):
    position = np.arange(seq_len, dtype=np.float32).reshape(-1, 1)
    i = np.arange(d_model) // 2
    exp_term = 2.0 * i / float(d_model)
    div_term = np.power(10000.0, exp_term).reshape(1, -1).astype(np.float32)
    pe = position / div_term
    pe[:, 0::2] = np.sin(pe[:, 0::2])
    pe[:, 1::2] = np.cos(pe[:, 1::2])
    return pe


_POS = jnp.asarray(_pos_encoding(_D)[: _L + 1], dtype=jnp.float32)


def _lane_bcast(v, r):
    idx = jnp.full((16,), r, dtype=jnp.int32)
    dnums = lax.GatherDimensionNumbers(
        offset_dims=(), collapsed_slice_dims=(0,), start_index_map=(0,)
    )
    return lax.gather(
        v, idx[:, None], dnums, (1,),
        mode=lax.GatherScatterMode.PROMISE_IN_BOUNDS,
    )


def _sc_body(data_h, mask_h, rv_h, pm_h, pos_h, out_h,
             rv_v, pm_v, mask_v, pos_sh, idx_v,
             rows0_v, rows1_v, rows2_v, pos0_v, pos1_v, pos2_v,
             semd0, semd1, semd2, semp0, semp1, semp2,
             semw0, semw1, semw2):
    cid = lax.axis_index("c")
    sid = lax.axis_index("s")
    b = sid
    t0w = cid * (_L // 2)

    data_b = data_h.at[b]
    out_b = out_h.at[b]

    pltpu.sync_copy(rv_h.at[b], rv_v)
    pltpu.sync_copy(pm_h.at[b], pm_v)
    pltpu.sync_copy(mask_h, mask_v)

    pltpu.sync_copy(
        pos_h.at[pl.ds(t0w + sid * 16, 16)], pos_sh.at[pl.ds(sid * 16, 16)]
    )
    plsc.subcore_barrier()

    iota = lax.iota(jnp.int32, 16)

    rows = (rows0_v, rows1_v, rows2_v)
    poss = (pos0_v, pos1_v, pos2_v)
    semd = (semd0, semd1, semd2)
    semp = (semp0, semp1, semp2)
    semw = (semw0, semw1, semw2)

    def indices_for(t):
        rv = plsc.load_gather(rv_v, [jnp.maximum(t - 1, 0)])
        j = jnp.where(t == 0, 0, rv + 1)
        pmv = plsc.load_gather(pm_v, [jnp.clip(j - 1, 0, _R - 1)])
        valid = (j == 0) | ((j <= _R) & (pmv == 1))
        src = jnp.where(valid, j, 0)
        return src, valid

    def issue_gathers(k, bi):
        src, valid = indices_for(t0w + 16 * k + iota)
        idx_v[bi, :] = src
        cpd = pltpu.async_copy(data_b.at[idx_v.at[bi]], rows[bi], semd[bi])
        cpp = pltpu.async_copy(pos_sh.at[pl.ds(16 * k, 16)], poss[bi], semp[bi])
        return valid, cpd, cpp

    def compute(valid, bi):
        rv_, pv_ = rows[bi], poss[bi]
        vf = jnp.where(valid, 1, 0)
        vbc = [_lane_bcast(vf, r) != 0 for r in range(16)]

        def slice_body(s, _):
            off = s * 16
            m = mask_v[pl.ds(off, 16)]
            for r in range(16):
                g = rv_[r, pl.ds(off, 16)]
                p = pv_[r, pl.ds(off, 16)]
                rv_[r, pl.ds(off, 16)] = jnp.where(vbc[r], g, m) + p
            return 0

        lax.fori_loop(0, _D // 16, slice_body, 0)

    pend = [None] * _NCHUNK
    wr = [None] * _NCHUNK
    pend[0] = issue_gathers(0, 0)
    for k in range(_NCHUNK):
        cur = k % _NBUF
        if k + 1 < _NCHUNK:
            if k >= 2:
                wr[k - 2].wait()
            pend[k + 1] = issue_gathers(k + 1, (k + 1) % _NBUF)
        valid, cpd, cpp = pend[k]
        cpd.wait()
        cpp.wait()
        compute(valid, cur)
        wr[k] = pltpu.async_copy(
            rows[cur], out_b.at[pl.ds(t0w + 16 * k, 16)], semw[cur]
        )

    wr[_NCHUNK - 3].wait()
    wr[_NCHUNK - 2].wait()
    wr[_NCHUNK - 1].wait()

    @pl.when(cid == 1)
    def _():
        t = jnp.full((16,), _L, jnp.int32)
        src, valid = indices_for(t)
        cpd = pltpu.async_copy(data_b.at[src], rows0_v, semd0)
        cpp = pltpu.async_copy(pos_h.at[t], pos0_v, semp0)
        cpd.wait()
        cpp.wait()
        compute(valid, 0)
        pltpu.sync_copy(rows0_v.at[pl.ds(0, 1)], out_b.at[pl.ds(_L, 1)])


@functools.partial(jax.jit, static_argnames=())
def _run(data, mask_token, revert_idx, padding_mask, pos):
    mesh = plsc.VectorSubcoreMesh(core_axis_name="c", subcore_axis_name="s")
    return pl.kernel(
        _sc_body,
        out_type=jax.ShapeDtypeStruct((_B, _L + 1, _D), jnp.float32),
        mesh=mesh,
        compiler_params=pltpu.CompilerParams(needs_layout_passes=False),
        scratch_types=[
            pltpu.VMEM((_L,), jnp.int32),
            pltpu.VMEM((_R,), jnp.int32),
            pltpu.VMEM((_D,), jnp.float32),
            pltpu.VMEM_SHARED((_L // 2, _D), jnp.float32),
            pltpu.VMEM((_NBUF, 16), jnp.int32),
            pltpu.VMEM((16, _D), jnp.float32),
            pltpu.VMEM((16, _D), jnp.float32),
            pltpu.VMEM((16, _D), jnp.float32),
            pltpu.VMEM((16, _D), jnp.float32),
            pltpu.VMEM((16, _D), jnp.float32),
            pltpu.VMEM((16, _D), jnp.float32),
            pltpu.SemaphoreType.DMA,
            pltpu.SemaphoreType.DMA,
            pltpu.SemaphoreType.DMA,
            pltpu.SemaphoreType.DMA,
            pltpu.SemaphoreType.DMA,
            pltpu.SemaphoreType.DMA,
            pltpu.SemaphoreType.DMA,
            pltpu.SemaphoreType.DMA,
            pltpu.SemaphoreType.DMA,
        ],
    )(data, mask_token, revert_idx, padding_mask, pos)


def kernel(data, mask_token, revert_idx, device, padding_mask):
    del device
    return _run(data, mask_token, revert_idx, padding_mask, _POS)
